# Initial kernel scaffold; baseline (speedup 1.0000x reference)
import jax
import jax.numpy as jnp
from jax.experimental import pallas as pl


def kernel(x, edge_index, W1, b1, W2, b2, Wm1, bm1, Wm2, bm2):
    raise NotImplementedError("write your pallas kernel here")


# speedup vs baseline: 6.7587x; 6.7587x over previous
import functools

import jax
import jax.numpy as jnp
from jax import lax
from jax.experimental import pallas as pl
from jax.experimental.pallas import tpu as pltpu
from jax.experimental.pallas import tpu_sc as plsc

NC = 2
NS = 16
DW = 16
CH = 128


def _mesh():
    return plsc.VectorSubcoreMesh(core_axis_name="c", subcore_axis_name="s")


def _pad_rows(n):
    m = 8 * NS
    return ((n + m - 1) // m) * m


def _sc_degrees(ei, n, d):
    e = ei.shape[1]
    ncht = e // CH
    nfull = ncht // NS
    extra = ncht - nfull * NS
    np_ = _pad_rows(n)
    rpt = np_ // NS

    @functools.partial(
        pl.kernel,
        out_type=(
            jax.ShapeDtypeStruct((np_, d), jnp.float32),
            jax.ShapeDtypeStruct((np_, d), jnp.float32),
        ),
        mesh=_mesh(),
        scratch_types=[
            pltpu.VMEM((2, CH), jnp.int32),
            pltpu.VMEM((CH, d), jnp.float32),
            pltpu.VMEM_SHARED((np_, d), jnp.float32),
        ],
    )
    def deg_kernel(ei_h, ones_h, zero_h, dego_h, degi_h, ei_v, ones_v, acc):
        c = lax.axis_index("c")
        s = lax.axis_index("s")
        pltpu.sync_copy(ones_h, ones_v)
        sl = pl.ds(s * rpt, rpt)
        pltpu.sync_copy(zero_h.at[sl], acc.at[sl])
        plsc.subcore_barrier()

        def step(chunk):
            pltpu.sync_copy(ei_h.at[:, pl.ds(chunk * CH, CH)], ei_v)

            @pl.when(c == 0)
            def _():
                pltpu.sync_copy(ones_v, acc.at[ei_v.at[0]], add=True)

            @pl.when(c == 1)
            def _():
                pltpu.sync_copy(ones_v, acc.at[ei_v.at[1]], add=True)

        def body(j, carry):
            step(j * NS + s)
            return carry
        lax.fori_loop(0, nfull, body, 0)
        if extra:
            @pl.when(s < extra)
            def _():
                step(nfull * NS + s)
        plsc.subcore_barrier()

        @pl.when(c == 0)
        def _():
            pltpu.sync_copy(acc.at[sl], dego_h.at[sl])

        @pl.when(c == 1)
        def _():
            pltpu.sync_copy(acc.at[sl], degi_h.at[sl])

    zeros = jnp.zeros((np_, d), jnp.float32)
    ones = jnp.ones((CH, d), jnp.float32)
    return deg_kernel(ei, ones, zeros)


def _sc_aggregate(h, ei):
    n, d = h.shape
    e = ei.shape[1]
    nw = NC * NS
    ncht = e // CH
    nfull = ncht // nw
    extra = ncht - nfull * nw
    np_ = _pad_rows(n)
    rpt = np_ // NS

    @functools.partial(
        pl.kernel,
        out_type=jax.ShapeDtypeStruct((NC, np_, d), jnp.float32),
        mesh=_mesh(),
        scratch_types=[
            pltpu.VMEM((2, CH), jnp.int32),
            pltpu.VMEM((CH, d), jnp.float32),
            pltpu.VMEM_SHARED((np_, d), jnp.float32),
            pltpu.SemaphoreType.DMA,
        ],
    )
    def agg_kernel(h_h, ei_h, zero_h, out_h, ei_v, rows, acc, sem):
        c = lax.axis_index("c")
        s = lax.axis_index("s")
        sl = pl.ds(s * rpt, rpt)
        pltpu.sync_copy(zero_h.at[sl], acc.at[sl])
        plsc.subcore_barrier()
        w = c * NS + s

        def step(chunk):
            pltpu.sync_copy(ei_h.at[:, pl.ds(chunk * CH, CH)], ei_v)
            pltpu.async_copy(h_h.at[ei_v.at[0]], rows, sem).wait()
            pltpu.sync_copy(rows, acc.at[ei_v.at[1]], add=True)

        def body(j, carry):
            step(j * nw + w)
            return carry
        lax.fori_loop(0, nfull, body, 0)
        if extra:
            @pl.when(w < extra)
            def _():
                step(nfull * nw + w)
        plsc.subcore_barrier()
        pltpu.sync_copy(acc.at[sl], out_h.at[c, sl])

    zeros = jnp.zeros((np_, d), jnp.float32)
    return agg_kernel(h, ei, zeros)


_BLK = 1000


def _norm_from(deg_ref, clip_lo=1.0):
    return lax.rsqrt(jnp.maximum(deg_ref[:, 0:1], clip_lo))


def _tc_scale(x, dego_p):
    n, d = x.shape

    def body(x_ref, dego_ref, o_ref):
        o_ref[...] = x_ref[...] * _norm_from(dego_ref)

    return pl.pallas_call(
        body,
        grid=(n // _BLK,),
        in_specs=[
            pl.BlockSpec((_BLK, d), lambda i: (i, 0)),
            pl.BlockSpec((_BLK, d), lambda i: (i, 0)),
        ],
        out_specs=pl.BlockSpec((_BLK, d), lambda i: (i, 0)),
        out_shape=jax.ShapeDtypeStruct((n, d), jnp.float32),
    )(x, dego_p)


def _tc_layer1(agg_p, degi_p, dego_p, W1, b1, W2, n):
    d = agg_p.shape[2]

    def body(p_ref, degi_ref, dego_ref, w1_ref, b1_ref, w2_ref, o_ref):
        a = (p_ref[0] + p_ref[1]) * _norm_from(degi_ref)
        h1 = jnp.maximum(
            jnp.dot(a, w1_ref[...], preferred_element_type=jnp.float32)
            + b1_ref[...][None, :], 0.0)
        g = h1 * _norm_from(dego_ref)
        o_ref[...] = jnp.dot(g, w2_ref[...], preferred_element_type=jnp.float32)

    return pl.pallas_call(
        body,
        grid=(n // _BLK,),
        in_specs=[
            pl.BlockSpec((NC, _BLK, d), lambda i: (0, i, 0)),
            pl.BlockSpec((_BLK, d), lambda i: (i, 0)),
            pl.BlockSpec((_BLK, d), lambda i: (i, 0)),
            pl.BlockSpec(W1.shape, lambda i: (0, 0)),
            pl.BlockSpec(b1.shape, lambda i: (0,)),
            pl.BlockSpec(W2.shape, lambda i: (0, 0)),
        ],
        out_specs=pl.BlockSpec((_BLK, W2.shape[1]), lambda i: (i, 0)),
        out_shape=jax.ShapeDtypeStruct((n, W2.shape[1]), jnp.float32),
    )(agg_p, degi_p, dego_p, W1, b1, W2)


def _tc_final(agg_p, degi_p, b2, Wm1, bm1, Wm2, bm2, n):
    d = agg_p.shape[2]

    def body(q_ref, degi_ref, b2_ref, wm1_ref, bm1_ref, wm2_ref, bm2_ref,
             out_ref, h2_ref):
        h2 = jnp.maximum(
            (q_ref[0] + q_ref[1]) * _norm_from(degi_ref)
            + b2_ref[...][None, :], 0.0)
        h2_ref[...] = h2
        t = jnp.maximum(
            jnp.dot(h2, wm1_ref[...], preferred_element_type=jnp.float32)
            + bm1_ref[...][None, :], 0.0)
        out_ref[...] = (
            jnp.dot(t, wm2_ref[...], preferred_element_type=jnp.float32)
            + bm2_ref[...][None, :])

    return pl.pallas_call(
        body,
        grid=(n // _BLK,),
        in_specs=[
            pl.BlockSpec((NC, _BLK, d), lambda i: (0, i, 0)),
            pl.BlockSpec((_BLK, d), lambda i: (i, 0)),
            pl.BlockSpec(b2.shape, lambda i: (0,)),
            pl.BlockSpec(Wm1.shape, lambda i: (0, 0)),
            pl.BlockSpec(bm1.shape, lambda i: (0,)),
            pl.BlockSpec(Wm2.shape, lambda i: (0, 0)),
            pl.BlockSpec(bm2.shape, lambda i: (0,)),
        ],
        out_specs=(
            pl.BlockSpec((_BLK, Wm2.shape[1]), lambda i: (i, 0)),
            pl.BlockSpec((_BLK, d), lambda i: (i, 0)),
        ),
        out_shape=(
            jax.ShapeDtypeStruct((n, Wm2.shape[1]), jnp.float32),
            jax.ShapeDtypeStruct((n, d), jnp.float32),
        ),
    )(agg_p, degi_p, b2, Wm1, bm1, Wm2, bm2)


def kernel(x, edge_index, W1, b1, W2, b2, Wm1, bm1, Wm2, bm2):
    n = x.shape[0]
    dego_p, degi_p = _sc_degrees(edge_index, n, x.shape[1])
    xs = _tc_scale(x, dego_p)
    agg1_p = _sc_aggregate(xs, edge_index)
    g = _tc_layer1(agg1_p, degi_p, dego_p, W1, b1, W2, n)
    agg2_p = _sc_aggregate(g, edge_index)
    out, h2 = _tc_final(agg2_p, degi_p, b2, Wm1, bm1, Wm2, bm2, n)
    return (out, h2)


# speedup vs baseline: 9.1222x; 1.3497x over previous
import functools

import jax
import jax.numpy as jnp
from jax import lax
from jax.experimental import pallas as pl
from jax.experimental.pallas import tpu as pltpu
from jax.experimental.pallas import tpu_sc as plsc

NC = 2
NS = 16
DW = 16
CH = 128


def _mesh():
    return plsc.VectorSubcoreMesh(core_axis_name="c", subcore_axis_name="s")


def _pad_rows(n):
    m = 8 * NS
    return ((n + m - 1) // m) * m


def _sc_degrees(ei, n, d):
    e = ei.shape[1]
    ncht = e // CH
    nfull = ncht // NS
    extra = ncht - nfull * NS
    np_ = _pad_rows(n)
    rpt = np_ // NS

    @functools.partial(
        pl.kernel,
        out_type=(
            jax.ShapeDtypeStruct((np_, d), jnp.float32),
            jax.ShapeDtypeStruct((np_, d), jnp.float32),
        ),
        mesh=_mesh(),
        scratch_types=[
            pltpu.VMEM((2, CH), jnp.int32),
            pltpu.VMEM((CH, d), jnp.float32),
            pltpu.VMEM_SHARED((np_, d), jnp.float32),
        ],
    )
    def deg_kernel(ei_h, ones_h, zero_h, dego_h, degi_h, ei_v, ones_v, acc):
        c = lax.axis_index("c")
        s = lax.axis_index("s")
        pltpu.sync_copy(ones_h, ones_v)
        sl = pl.ds(s * rpt, rpt)
        pltpu.sync_copy(zero_h.at[sl], acc.at[sl])
        plsc.subcore_barrier()

        def step(chunk):
            pltpu.sync_copy(ei_h.at[:, pl.ds(chunk * CH, CH)], ei_v)

            @pl.when(c == 0)
            def _():
                pltpu.sync_copy(ones_v, acc.at[ei_v.at[0]], add=True)

            @pl.when(c == 1)
            def _():
                pltpu.sync_copy(ones_v, acc.at[ei_v.at[1]], add=True)

        def body(j, carry):
            step(j * NS + s)
            return carry
        lax.fori_loop(0, nfull, body, 0)
        if extra:
            @pl.when(s < extra)
            def _():
                step(nfull * NS + s)
        plsc.subcore_barrier()

        @pl.when(c == 0)
        def _():
            pltpu.sync_copy(acc.at[sl], dego_h.at[sl])

        @pl.when(c == 1)
        def _():
            pltpu.sync_copy(acc.at[sl], degi_h.at[sl])

    zeros = jnp.zeros((np_, d), jnp.float32)
    ones = jnp.ones((CH, d), jnp.float32)
    return deg_kernel(ei, ones, zeros)


def _sc_aggregate(h, ei):
    n, d = h.shape
    e = ei.shape[1]
    nw = NC * NS
    ncht = e // CH
    nfull = ncht // nw
    extra = ncht - nfull * nw
    np_ = _pad_rows(n)
    rpt = np_ // NS

    @functools.partial(
        pl.kernel,
        out_type=jax.ShapeDtypeStruct((NC, np_, d), jnp.float32),
        mesh=_mesh(),
        scratch_types=[
            pltpu.VMEM((2, 2, CH), jnp.int32),
            pltpu.VMEM((2, CH, d), jnp.float32),
            pltpu.VMEM_SHARED((np_, d), jnp.float32),
            pltpu.SemaphoreType.DMA,
            pltpu.SemaphoreType.DMA,
        ],
    )
    def agg_kernel(h_h, ei_h, zero_h, out_h, ei_v, rows, acc, sem0, sem1):
        c = lax.axis_index("c")
        s = lax.axis_index("s")
        sems = (sem0, sem1)
        sl = pl.ds(s * rpt, rpt)
        pltpu.sync_copy(zero_h.at[sl], acc.at[sl])
        plsc.subcore_barrier()
        w = c * NS + s

        def fetch(chunk, b):
            pltpu.sync_copy(ei_h.at[:, pl.ds(chunk * CH, CH)], ei_v.at[b])
            pltpu.async_copy(h_h.at[ei_v.at[b, 0]], rows.at[b], sems[b])

        def drain(b):
            pltpu.make_async_copy(h_h.at[ei_v.at[b, 0]], rows.at[b],
                                  sems[b]).wait()
            pltpu.sync_copy(rows.at[b], acc.at[ei_v.at[b, 1]], add=True)

        for b in range(2):
            fetch(b * nw + w, b)

        def body(j2, carry):
            for b in range(2):
                j = 2 * j2 + b
                drain(b)

                @pl.when(j + 2 < nfull)
                def _():
                    fetch((j + 2) * nw + w, b)
            return carry
        lax.fori_loop(0, nfull // 2, body, 0)
        if nfull % 2:
            drain(0)
        if extra:
            @pl.when(w < extra)
            def _():
                fetch(nfull * nw + w, 0)
                drain(0)
        plsc.subcore_barrier()
        pltpu.sync_copy(acc.at[sl], out_h.at[c, sl])

    zeros = jnp.zeros((np_, d), jnp.float32)
    return agg_kernel(h, ei, zeros)


_BLK = 1000


def _norm_from(deg_ref, clip_lo=1.0):
    return lax.rsqrt(jnp.maximum(deg_ref[:, 0:1], clip_lo))


def _tc_scale(x, dego_p):
    n, d = x.shape

    def body(x_ref, dego_ref, o_ref):
        o_ref[...] = x_ref[...] * _norm_from(dego_ref)

    return pl.pallas_call(
        body,
        grid=(n // _BLK,),
        in_specs=[
            pl.BlockSpec((_BLK, d), lambda i: (i, 0)),
            pl.BlockSpec((_BLK, d), lambda i: (i, 0)),
        ],
        out_specs=pl.BlockSpec((_BLK, d), lambda i: (i, 0)),
        out_shape=jax.ShapeDtypeStruct((n, d), jnp.float32),
    )(x, dego_p)


def _tc_layer1(agg_p, degi_p, dego_p, W1, b1, W2, n):
    d = agg_p.shape[2]

    def body(p_ref, degi_ref, dego_ref, w1_ref, b1_ref, w2_ref, o_ref):
        a = (p_ref[0] + p_ref[1]) * _norm_from(degi_ref)
        h1 = jnp.maximum(
            jnp.dot(a, w1_ref[...], preferred_element_type=jnp.float32)
            + b1_ref[...][None, :], 0.0)
        g = h1 * _norm_from(dego_ref)
        o_ref[...] = jnp.dot(g, w2_ref[...], preferred_element_type=jnp.float32)

    return pl.pallas_call(
        body,
        grid=(n // _BLK,),
        in_specs=[
            pl.BlockSpec((NC, _BLK, d), lambda i: (0, i, 0)),
            pl.BlockSpec((_BLK, d), lambda i: (i, 0)),
            pl.BlockSpec((_BLK, d), lambda i: (i, 0)),
            pl.BlockSpec(W1.shape, lambda i: (0, 0)),
            pl.BlockSpec(b1.shape, lambda i: (0,)),
            pl.BlockSpec(W2.shape, lambda i: (0, 0)),
        ],
        out_specs=pl.BlockSpec((_BLK, W2.shape[1]), lambda i: (i, 0)),
        out_shape=jax.ShapeDtypeStruct((n, W2.shape[1]), jnp.float32),
    )(agg_p, degi_p, dego_p, W1, b1, W2)


def _tc_final(agg_p, degi_p, b2, Wm1, bm1, Wm2, bm2, n):
    d = agg_p.shape[2]

    def body(q_ref, degi_ref, b2_ref, wm1_ref, bm1_ref, wm2_ref, bm2_ref,
             out_ref, h2_ref):
        h2 = jnp.maximum(
            (q_ref[0] + q_ref[1]) * _norm_from(degi_ref)
            + b2_ref[...][None, :], 0.0)
        h2_ref[...] = h2
        t = jnp.maximum(
            jnp.dot(h2, wm1_ref[...], preferred_element_type=jnp.float32)
            + bm1_ref[...][None, :], 0.0)
        out_ref[...] = (
            jnp.dot(t, wm2_ref[...], preferred_element_type=jnp.float32)
            + bm2_ref[...][None, :])

    return pl.pallas_call(
        body,
        grid=(n // _BLK,),
        in_specs=[
            pl.BlockSpec((NC, _BLK, d), lambda i: (0, i, 0)),
            pl.BlockSpec((_BLK, d), lambda i: (i, 0)),
            pl.BlockSpec(b2.shape, lambda i: (0,)),
            pl.BlockSpec(Wm1.shape, lambda i: (0, 0)),
            pl.BlockSpec(bm1.shape, lambda i: (0,)),
            pl.BlockSpec(Wm2.shape, lambda i: (0, 0)),
            pl.BlockSpec(bm2.shape, lambda i: (0,)),
        ],
        out_specs=(
            pl.BlockSpec((_BLK, Wm2.shape[1]), lambda i: (i, 0)),
            pl.BlockSpec((_BLK, d), lambda i: (i, 0)),
        ),
        out_shape=(
            jax.ShapeDtypeStruct((n, Wm2.shape[1]), jnp.float32),
            jax.ShapeDtypeStruct((n, d), jnp.float32),
        ),
    )(agg_p, degi_p, b2, Wm1, bm1, Wm2, bm2)


def kernel(x, edge_index, W1, b1, W2, b2, Wm1, bm1, Wm2, bm2):
    n = x.shape[0]
    dego_p, degi_p = _sc_degrees(edge_index, n, x.shape[1])
    xs = _tc_scale(x, dego_p)
    agg1_p = _sc_aggregate(xs, edge_index)
    g = _tc_layer1(agg1_p, degi_p, dego_p, W1, b1, W2, n)
    agg2_p = _sc_aggregate(g, edge_index)
    out, h2 = _tc_final(agg2_p, degi_p, b2, Wm1, bm1, Wm2, bm2, n)
    return (out, h2)


# speedup vs baseline: 10.6494x; 1.1674x over previous
import functools

import jax
import jax.numpy as jnp
from jax import lax
from jax.experimental import pallas as pl
from jax.experimental.pallas import tpu as pltpu
from jax.experimental.pallas import tpu_sc as plsc

NC = 2
NS = 16
DW = 16
CH = 128


def _mesh():
    return plsc.VectorSubcoreMesh(core_axis_name="c", subcore_axis_name="s")


def _pad_rows(n):
    m = 8 * NS
    return ((n + m - 1) // m) * m


def _sc_degrees(ei, n, d):
    e = ei.shape[1]
    ncht = e // CH
    nfull = ncht // NS
    extra = ncht - nfull * NS
    nmain = (nfull // 4) * 4
    np_ = _pad_rows(n)
    rpt = np_ // NS

    @functools.partial(
        pl.kernel,
        out_type=(
            jax.ShapeDtypeStruct((np_, d), jnp.float32),
            jax.ShapeDtypeStruct((np_, d), jnp.float32),
        ),
        mesh=_mesh(),
        scratch_types=[
            pltpu.VMEM((4, 1, CH), jnp.int32),
            pltpu.VMEM((CH, d), jnp.float32),
            pltpu.VMEM_SHARED((np_, d), jnp.float32),
            [pltpu.SemaphoreType.DMA] * 4,
            [pltpu.SemaphoreType.DMA] * 4,
        ],
    )
    def deg_kernel(ei_h, ones_h, zero_h, dego_h, degi_h,
                   ei_v, ones_v, acc, isems, ssems):
        c = lax.axis_index("c")
        s = lax.axis_index("s")
        pltpu.sync_copy(ones_h, ones_v)
        sl = pl.ds(s * rpt, rpt)
        pltpu.sync_copy(zero_h.at[sl], acc.at[sl])
        plsc.subcore_barrier()

        def run(row):
            def fetch(chunk, b):
                pltpu.async_copy(
                    ei_h.at[row:row + 1, pl.ds(chunk * CH, CH)],
                    ei_v.at[b], isems[b])

            def wait_fetch(b):
                pltpu.make_async_copy(
                    ei_h.at[row:row + 1, pl.ds(0, CH)],
                    ei_v.at[b], isems[b]).wait()

            def start_scatter(b):
                pltpu.async_copy(ones_v, acc.at[ei_v.at[b, 0]], ssems[b],
                                 add=True)

            def wait_scatter(b):
                pltpu.make_async_copy(ones_v, acc.at[ei_v.at[b, 0]],
                                      ssems[b]).wait()

            for b in range(2):
                fetch((b * NS + s), b)

            def body(jj, carry):
                for b4 in range(4):
                    j = 4 * jj + b4
                    wait_fetch(b4)
                    start_scatter(b4)
                    b2 = (b4 + 2) % 4

                    @pl.when(j >= 2)
                    def _():
                        wait_scatter(b2)

                    @pl.when(j + 2 < nmain)
                    def _():
                        fetch((j + 2) * NS + s, b2)
                return carry
            lax.fori_loop(0, nmain // 4, body, 0)
            wait_scatter((nmain - 2) % 4)
            wait_scatter((nmain - 1) % 4)
            for j in range(nmain, nfull):
                fetch(j * NS + s, 0)
                wait_fetch(0)
                start_scatter(0)
                wait_scatter(0)
            if extra:
                @pl.when(s < extra)
                def _():
                    fetch(nfull * NS + s, 0)
                    wait_fetch(0)
                    start_scatter(0)
                    wait_scatter(0)

        @pl.when(c == 0)
        def _():
            run(0)

        @pl.when(c == 1)
        def _():
            run(1)
        plsc.subcore_barrier()

        @pl.when(c == 0)
        def _():
            pltpu.sync_copy(acc.at[sl], dego_h.at[sl])

        @pl.when(c == 1)
        def _():
            pltpu.sync_copy(acc.at[sl], degi_h.at[sl])

    zeros = jnp.zeros((np_, d), jnp.float32)
    ones = jnp.ones((CH, d), jnp.float32)
    return deg_kernel(ei, ones, zeros)


def _sc_aggregate(h, ei):
    n, d = h.shape
    e = ei.shape[1]
    nw = NC * NS
    ncht = e // CH
    nfull = ncht // nw
    extra = ncht - nfull * nw
    np_ = _pad_rows(n)
    rpt = np_ // NS

    @functools.partial(
        pl.kernel,
        out_type=jax.ShapeDtypeStruct((NC, np_, d), jnp.float32),
        mesh=_mesh(),
        scratch_types=[
            pltpu.VMEM((2, 2, CH), jnp.int32),
            pltpu.VMEM((2, CH, d), jnp.float32),
            pltpu.VMEM_SHARED((np_, d), jnp.float32),
            pltpu.SemaphoreType.DMA,
            pltpu.SemaphoreType.DMA,
        ],
    )
    def agg_kernel(h_h, ei_h, zero_h, out_h, ei_v, rows, acc, sem0, sem1):
        c = lax.axis_index("c")
        s = lax.axis_index("s")
        sems = (sem0, sem1)
        sl = pl.ds(s * rpt, rpt)
        pltpu.sync_copy(zero_h.at[sl], acc.at[sl])
        plsc.subcore_barrier()
        w = c * NS + s

        def fetch(chunk, b):
            pltpu.sync_copy(ei_h.at[:, pl.ds(chunk * CH, CH)], ei_v.at[b])
            pltpu.async_copy(h_h.at[ei_v.at[b, 0]], rows.at[b], sems[b])

        def drain(b):
            pltpu.make_async_copy(h_h.at[ei_v.at[b, 0]], rows.at[b],
                                  sems[b]).wait()
            pltpu.sync_copy(rows.at[b], acc.at[ei_v.at[b, 1]], add=True)

        for b in range(2):
            fetch(b * nw + w, b)

        def body(j2, carry):
            for b in range(2):
                j = 2 * j2 + b
                drain(b)

                @pl.when(j + 2 < nfull)
                def _():
                    fetch((j + 2) * nw + w, b)
            return carry
        lax.fori_loop(0, nfull // 2, body, 0)
        if nfull % 2:
            drain(0)
        if extra:
            @pl.when(w < extra)
            def _():
                fetch(nfull * nw + w, 0)
                drain(0)
        plsc.subcore_barrier()
        pltpu.sync_copy(acc.at[sl], out_h.at[c, sl])

    zeros = jnp.zeros((np_, d), jnp.float32)
    return agg_kernel(h, ei, zeros)


_BLK = 1000


def _norm_from(deg_ref, clip_lo=1.0):
    return lax.rsqrt(jnp.maximum(deg_ref[:, 0:1], clip_lo))


def _tc_scale(x, dego_p):
    n, d = x.shape

    def body(x_ref, dego_ref, o_ref):
        o_ref[...] = x_ref[...] * _norm_from(dego_ref)

    return pl.pallas_call(
        body,
        grid=(n // _BLK,),
        in_specs=[
            pl.BlockSpec((_BLK, d), lambda i: (i, 0)),
            pl.BlockSpec((_BLK, d), lambda i: (i, 0)),
        ],
        out_specs=pl.BlockSpec((_BLK, d), lambda i: (i, 0)),
        out_shape=jax.ShapeDtypeStruct((n, d), jnp.float32),
    )(x, dego_p)


def _tc_layer1(agg_p, degi_p, dego_p, W1, b1, W2, n):
    d = agg_p.shape[2]

    def body(p_ref, degi_ref, dego_ref, w1_ref, b1_ref, w2_ref, o_ref):
        a = (p_ref[0] + p_ref[1]) * _norm_from(degi_ref)
        h1 = jnp.maximum(
            jnp.dot(a, w1_ref[...], preferred_element_type=jnp.float32)
            + b1_ref[...][None, :], 0.0)
        g = h1 * _norm_from(dego_ref)
        o_ref[...] = jnp.dot(g, w2_ref[...], preferred_element_type=jnp.float32)

    return pl.pallas_call(
        body,
        grid=(n // _BLK,),
        in_specs=[
            pl.BlockSpec((NC, _BLK, d), lambda i: (0, i, 0)),
            pl.BlockSpec((_BLK, d), lambda i: (i, 0)),
            pl.BlockSpec((_BLK, d), lambda i: (i, 0)),
            pl.BlockSpec(W1.shape, lambda i: (0, 0)),
            pl.BlockSpec(b1.shape, lambda i: (0,)),
            pl.BlockSpec(W2.shape, lambda i: (0, 0)),
        ],
        out_specs=pl.BlockSpec((_BLK, W2.shape[1]), lambda i: (i, 0)),
        out_shape=jax.ShapeDtypeStruct((n, W2.shape[1]), jnp.float32),
    )(agg_p, degi_p, dego_p, W1, b1, W2)


def _tc_final(agg_p, degi_p, b2, Wm1, bm1, Wm2, bm2, n):
    d = agg_p.shape[2]

    def body(q_ref, degi_ref, b2_ref, wm1_ref, bm1_ref, wm2_ref, bm2_ref,
             out_ref, h2_ref):
        h2 = jnp.maximum(
            (q_ref[0] + q_ref[1]) * _norm_from(degi_ref)
            + b2_ref[...][None, :], 0.0)
        h2_ref[...] = h2
        t = jnp.maximum(
            jnp.dot(h2, wm1_ref[...], preferred_element_type=jnp.float32)
            + bm1_ref[...][None, :], 0.0)
        out_ref[...] = (
            jnp.dot(t, wm2_ref[...], preferred_element_type=jnp.float32)
            + bm2_ref[...][None, :])

    return pl.pallas_call(
        body,
        grid=(n // _BLK,),
        in_specs=[
            pl.BlockSpec((NC, _BLK, d), lambda i: (0, i, 0)),
            pl.BlockSpec((_BLK, d), lambda i: (i, 0)),
            pl.BlockSpec(b2.shape, lambda i: (0,)),
            pl.BlockSpec(Wm1.shape, lambda i: (0, 0)),
            pl.BlockSpec(bm1.shape, lambda i: (0,)),
            pl.BlockSpec(Wm2.shape, lambda i: (0, 0)),
            pl.BlockSpec(bm2.shape, lambda i: (0,)),
        ],
        out_specs=(
            pl.BlockSpec((_BLK, Wm2.shape[1]), lambda i: (i, 0)),
            pl.BlockSpec((_BLK, d), lambda i: (i, 0)),
        ),
        out_shape=(
            jax.ShapeDtypeStruct((n, Wm2.shape[1]), jnp.float32),
            jax.ShapeDtypeStruct((n, d), jnp.float32),
        ),
    )(agg_p, degi_p, b2, Wm1, bm1, Wm2, bm2)


def kernel(x, edge_index, W1, b1, W2, b2, Wm1, bm1, Wm2, bm2):
    n = x.shape[0]
    dego_p, degi_p = _sc_degrees(edge_index, n, x.shape[1])
    xs = _tc_scale(x, dego_p)
    agg1_p = _sc_aggregate(xs, edge_index)
    g = _tc_layer1(agg1_p, degi_p, dego_p, W1, b1, W2, n)
    agg2_p = _sc_aggregate(g, edge_index)
    out, h2 = _tc_final(agg2_p, degi_p, b2, Wm1, bm1, Wm2, bm2, n)
    return (out, h2)
